# baseline (speedup 1.0000x reference)
import functools

import jax
import jax.numpy as jnp
from jax import lax
from jax.experimental import pallas as pl
from jax.experimental.pallas import tpu as pltpu
from jax.experimental.pallas import tpu_sc as plsc

_ROWS = 4096
_COLS = 16384
_NSEL = 64

_NC = 2
_NS = 16
_LANES = 16
_NW = _NC * _NS
_SLICE = _ROWS // _NW


def _sign_body(sel_hbm, out_hbm, sel_v, sign_v):
    wid = lax.axis_index("s") * _NC + lax.axis_index("c")
    base = wid * _SLICE
    pltpu.sync_copy(sel_hbm, sel_v)
    ones = jnp.ones((_LANES,), jnp.float32)
    for i in range(_SLICE // _LANES):
        sign_v[pl.ds(i * _LANES, _LANES)] = ones
    neg = -ones
    for g in range(_NSEL // _LANES):
        idx = sel_v[pl.ds(g * _LANES, _LANES)]
        mask = (idx >= base) & (idx < base + _SLICE)
        local = jnp.where(mask, idx - base, 0)
        plsc.store_scatter(sign_v, [local], neg, mask=mask)
    pltpu.sync_copy(sign_v, out_hbm.at[pl.ds(base, _SLICE)])


def _make_sign(selection):
    mesh = plsc.VectorSubcoreMesh(core_axis_name="c", subcore_axis_name="s")
    return pl.kernel(
        _sign_body,
        out_type=jax.ShapeDtypeStruct((_ROWS,), jnp.float32),
        mesh=mesh,
        scratch_types=[
            pltpu.VMEM((_NSEL,), jnp.int32),
            pltpu.VMEM((_SLICE,), jnp.float32),
        ],
        compiler_params=pltpu.CompilerParams(needs_layout_passes=False),
    )(selection)


def _flip_body(d_ref, s_ref, o_ref):
    o_ref[...] = d_ref[...] * s_ref[...]


def _flip(data, sign2d, block_rows):
    grid = (_ROWS // block_rows,)
    return pl.pallas_call(
        _flip_body,
        grid=grid,
        in_specs=[
            pl.BlockSpec((block_rows, _COLS), lambda i: (i, 0)),
            pl.BlockSpec((block_rows, 1), lambda i: (i, 0)),
        ],
        out_specs=pl.BlockSpec((block_rows, _COLS), lambda i: (i, 0)),
        out_shape=jax.ShapeDtypeStruct((_ROWS, _COLS), jnp.float32),
    )(data, sign2d)


def kernel(data, selection):
    sel = selection.astype(jnp.int32)
    sign = _make_sign(sel)
    return _flip(data, sign[:, None], block_rows=128)


# speedup vs baseline: 1.1416x; 1.1416x over previous
import jax
import jax.numpy as jnp
from jax.experimental import pallas as pl

_ROWS = 4096
_COLS = 16384
_NSEL = 64


def _flip_body(block_rows, sel_ref, d_ref, o_ref):
    i = pl.program_id(0)
    rows = jax.lax.broadcasted_iota(jnp.int32, (block_rows, _NSEL), 0) + i * block_rows
    match = (rows == sel_ref[...]).any(axis=1, keepdims=True)
    sign = jnp.where(match, -1.0, 1.0).astype(jnp.float32)
    o_ref[...] = d_ref[...] * sign


def kernel(data, selection):
    sel = selection.astype(jnp.int32).reshape(1, _NSEL)
    block_rows = 128
    grid = (_ROWS // block_rows,)
    import functools
    return pl.pallas_call(
        functools.partial(_flip_body, block_rows),
        grid=grid,
        in_specs=[
            pl.BlockSpec((1, _NSEL), lambda i: (0, 0)),
            pl.BlockSpec((block_rows, _COLS), lambda i: (i, 0)),
        ],
        out_specs=pl.BlockSpec((block_rows, _COLS), lambda i: (i, 0)),
        out_shape=jax.ShapeDtypeStruct((_ROWS, _COLS), jnp.float32),
    )(sel, data)
